# baseline (speedup 1.0000x reference)
import functools

import jax
import jax.numpy as jnp
from jax import lax
from jax.experimental import pallas as pl
from jax.experimental.pallas import tpu as pltpu
from jax.experimental.pallas import tpu_sc as plsc

KEY_DIM = 4096
PFC_DIM = 1024
N_PATCHES = 4
N_SLOTS = 1024
EPS = 8
D_MEM = PFC_DIM + N_PATCHES * 3
IN_DIM = KEY_DIM + PFC_DIM
H_DIM = KEY_DIM * 2

RSC = 2560
RTC = H_DIM - RSC

BM = 256
NB_A = RTC // BM
NB_B = KEY_DIM // BM
NB_C = N_SLOTS // BM
NRING = 5

NW = 32
ROWS_W = RSC // NW
CH = 8
NCH = ROWS_W // CH

_NEG = float('-inf')


def _sc_h_body(w0_hbm, c_hbm, hp_hbm, cbuf, wchunk, obuf, sems):
    wid = lax.axis_index("s") * 2 + lax.axis_index("c")
    base = wid * ROWS_W

    def chunk_copy(ch, par):
        return pltpu.make_async_copy(
            w0_hbm.at[pl.ds(RTC + base + ch * CH, CH), :],
            wchunk.at[par], sems.at[par])

    chunk_copy(0, 0).start()
    pltpu.sync_copy(c_hbm, cbuf)

    for ch in range(NCH):
        par = ch % 2
        if ch + 1 < NCH:
            chunk_copy(ch + 1, (ch + 1) % 2).start()
        chunk_copy(ch, par).wait()
        for r in range(CH):
            def jbody(j, accs):
                a0, a1, a2, a3 = accs
                b = j * 64
                a0 = a0 + wchunk[par, r, pl.ds(b, 16)] * cbuf[pl.ds(b, 16)]
                a1 = a1 + wchunk[par, r, pl.ds(b + 16, 16)] \
                    * cbuf[pl.ds(b + 16, 16)]
                a2 = a2 + wchunk[par, r, pl.ds(b + 32, 16)] \
                    * cbuf[pl.ds(b + 32, 16)]
                a3 = a3 + wchunk[par, r, pl.ds(b + 48, 16)] \
                    * cbuf[pl.ds(b + 48, 16)]
                return (a0, a1, a2, a3)

            z = jnp.zeros((16,), jnp.float32)
            a0, a1, a2, a3 = lax.fori_loop(0, IN_DIM // 64, jbody,
                                           (z, z, z, z))
            obuf[pl.ds((ch * CH + r) * 16, 16)] = (a0 + a1) + (a2 + a3)

    pltpu.sync_copy(obuf, hp_hbm.at[pl.ds(base * 16, ROWS_W * 16)])


def _make_sc_h_kernel():
    return pl.kernel(
        _sc_h_body,
        mesh=plsc.VectorSubcoreMesh(core_axis_name="c", subcore_axis_name="s"),
        out_type=jax.ShapeDtypeStruct((RSC * 16,), jnp.float32),
        scratch_types=[
            pltpu.VMEM((IN_DIM,), jnp.float32),
            pltpu.VMEM((2, CH, IN_DIM), jnp.float32),
            pltpu.VMEM((ROWS_W * 16,), jnp.float32),
            pltpu.SemaphoreType.DMA((2,)),
        ])


def _tc1_kernel(comb_ref, b0_ref, w0_hbm, h_ref, wbuf, sems):

    def w0_copy(i, p):
        return pltpu.make_async_copy(
            w0_hbm.at[pl.ds(i * BM, BM), :], wbuf.at[p], sems.at[p])

    for g in range(NRING - 1):
        w0_copy(g, g % NRING).start()

    comb = comb_ref[...]

    def body_a(i, _):
        p = lax.rem(i, NRING)

        @pl.when(i + NRING - 1 < NB_A)
        def _():
            w0_copy(i + NRING - 1, lax.rem(i + NRING - 1, NRING)).start()

        w0_copy(i, p).wait()
        w = wbuf[p]
        col = jnp.sum(w * comb, axis=1, keepdims=True)
        h_ref[:, pl.ds(i * BM, BM)] = jnp.maximum(
            col.T + b0_ref[:, pl.ds(i * BM, BM)], 0.0)
        return 0

    lax.fori_loop(0, NB_A, body_a, 0)


def _tc2_kernel(htc_ref, hp_ref, b0sc_ref, b2_ref, pfc_row_ref, pfc_col_ref,
                tde_ref, wg1_ref, bg1_ref, wg2_ref, bg2_ref, br_ref,
                wn_ref, bn_ref,
                w2_hbm, proto_hbm, ep_hbm, td_hbm, cnt_hbm, wr_hbm,
                newpfc_ref, alpha_ref, onehot_ref, nm_ref,
                wbuf, wrbuf, epbuf, tdbuf, cntbuf, hsc_row, key_row,
                best_val, best_idx, sems, wrsem):

    NB2 = NB_B + NB_C

    def w2_copy(i, p):
        return pltpu.make_async_copy(
            w2_hbm.at[pl.ds(i * BM, BM), :], wbuf.at[p], sems.at[p])

    def proto_copy(i, p):
        return pltpu.make_async_copy(
            proto_hbm.at[pl.ds(i * BM, BM), :],
            wbuf.at[p, :, pl.ds(0, KEY_DIM)], sems.at[p])

    def start_block(g):
        p = lax.rem(g, NRING)

        @pl.when(g < NB_B)
        def _():
            w2_copy(g, p).start()

        @pl.when(jnp.logical_and(g >= NB_B, g < NB2))
        def _():
            proto_copy(g - NB_B, p).start()

    pltpu.make_async_copy(wr_hbm, wrbuf, wrsem).start()
    for g in range(NRING - 1):
        w2_copy(g, g % NRING).start()

    htc = htc_ref[...]

    for j in range(RSC // BM):
        colp = jnp.sum(hp_ref[pl.ds(j * BM, BM), :], axis=1, keepdims=True)
        hsc_row[:, pl.ds(j * BM, BM)] = jnp.maximum(
            colp.T + b0sc_ref[:, pl.ds(j * BM, BM)], 0.0)
    hsc = hsc_row[...]

    def body_b(i, _):
        p = lax.rem(i, NRING)
        start_block(i + NRING - 1)
        w2_copy(i, p).wait()
        col = jnp.sum(wbuf[p, :, pl.ds(0, RTC)] * htc, axis=1, keepdims=True) \
            + jnp.sum(wbuf[p, :, pl.ds(RTC, RSC)] * hsc, axis=1, keepdims=True)
        key_row[:, pl.ds(i * BM, BM)] = col.T + b2_ref[:, pl.ds(i * BM, BM)]
        return 0

    lax.fori_loop(0, NB_B, body_b, 0)

    best_val[0] = _NEG
    best_idx[0] = 0
    key = key_row[...]
    knorm = jnp.clip(jnp.sqrt(jnp.sum(key * key)), 1e-12, None)

    def body_c(i, _):
        g = NB_B + i
        p = lax.rem(g, NRING)
        proto_copy(i, p).wait()
        w = wbuf[p, :, pl.ds(0, KEY_DIM)]
        raw = jnp.sum(w * key, axis=1, keepdims=True)
        pn = jnp.clip(jnp.sqrt(jnp.sum(w * w, axis=1, keepdims=True)),
                      1e-12, None)
        sims = raw / (pn * knorm)
        mx = jnp.max(sims)
        idx = lax.broadcasted_iota(jnp.int32, (BM, 1), 0) + i * BM
        bidx = jnp.min(jnp.where(sims == mx, idx, jnp.int32(2**30)))

        @pl.when(mx > best_val[0])
        def _():
            best_val[0] = mx
            best_idx[0] = bidx

        return 0

    lax.fori_loop(0, NB_C, body_c, 0)

    slot = best_idx[0]
    ep_cp = pltpu.make_async_copy(ep_hbm.at[pl.ds(slot, 1)], epbuf, sems.at[0])
    td_cp = pltpu.make_async_copy(td_hbm.at[pl.ds(slot, 1)], tdbuf, sems.at[1])
    cnt_cp = pltpu.make_async_copy(cnt_hbm.at[pl.ds(slot, 1)], cntbuf,
                                   sems.at[2])
    ep_cp.start()
    td_cp.start()
    cnt_cp.start()
    ep_cp.wait()
    td_cp.wait()
    cnt_cp.wait()
    pltpu.make_async_copy(wr_hbm, wrbuf, wrsem).wait()

    eps = epbuf[0]
    stored = eps[:, :PFC_DIM]
    pfc_row = pfc_row_ref[...]
    pnorm = jnp.clip(jnp.sqrt(jnp.sum(pfc_row * pfc_row)), 1e-12, None)
    pn = pfc_row / pnorm
    snorm = jnp.clip(jnp.sqrt(jnp.sum(stored * stored, axis=1, keepdims=True)),
                     1e-12, None)
    sims_e = jnp.sum(stored * pn, axis=1, keepdims=True) / snorm

    td = tdbuf[0]
    rel = sims_e * jnp.clip(jnp.abs(td), 1e-6, None)
    n_eps = jnp.minimum(cntbuf[0], EPS)
    idx8 = lax.broadcasted_iota(jnp.int32, (EPS, 1), 0)
    rel = jnp.where(idx8 < n_eps, rel, _NEG)
    mx = jnp.max(rel, axis=0, keepdims=True)
    bidx = jnp.min(jnp.where(rel == mx, idx8, jnp.int32(2**30)),
                   axis=0, keepdims=True)
    oh8 = (idx8 == bidx).astype(jnp.float32)
    ep_content = jnp.sum(eps * oh8, axis=0, keepdims=True)
    ep_td = jnp.sum(td * oh8, axis=0, keepdims=True)

    wg1 = wg1_ref[...]
    x1 = jnp.abs(tde_ref[...])
    g = jnp.tanh(wg1[:, 0:1] * best_val[0] + wg1[:, 1:2] * x1
                 + wg1[:, 2:3] * ep_td + bg1_ref[...])
    alpha = jnp.tanh(jnp.sum(wg2_ref[...] * g, axis=0, keepdims=True)
                     + bg2_ref[...])

    delta = jnp.sum(wrbuf[...] * ep_content, axis=1, keepdims=True) \
        + br_ref[...]
    newpfc_ref[...] = pfc_col_ref[...] + alpha * delta
    alpha_ref[...] = alpha

    ii = lax.broadcasted_iota(jnp.int32, (N_SLOTS, 1), 0)
    onehot_ref[...] = (ii == slot).astype(jnp.float32)

    nm = jnp.sum(wn_ref[...] * ep_content, axis=1, keepdims=True) + bn_ref[...]
    rows = lax.broadcasted_iota(jnp.int32, (3 * N_PATCHES, 1), 0)
    hi = jnp.where(rows < 2 * N_PATCHES, 1.0, 0.5)
    nm_ref[...] = jnp.clip(nm, 0.1, hi)


def kernel(activation_summary, pfc_state, current_td_error, prototypes,
           log_temperature, W0, b0, W2, b2, episodes, ep_td_errors, ep_count,
           Wg1, bg1, Wg2, bg2, Wr, br, Wn, bn):
    f32 = jnp.float32
    combined = jnp.concatenate(
        [activation_summary.reshape(1, KEY_DIM), pfc_state], axis=1)

    h_sc = _make_sc_h_kernel()(W0, combined.reshape(IN_DIM))

    vm = pl.BlockSpec(memory_space=pl.ANY)
    h_tc = pl.pallas_call(
        _tc1_kernel,
        in_specs=[pl.BlockSpec(memory_space=pltpu.VMEM),
                  pl.BlockSpec(memory_space=pltpu.VMEM), vm],
        out_specs=pl.BlockSpec(memory_space=pltpu.VMEM),
        out_shape=jax.ShapeDtypeStruct((1, RTC), f32),
        scratch_shapes=[
            pltpu.VMEM((NRING, BM, IN_DIM), f32),
            pltpu.SemaphoreType.DMA((NRING,)),
        ],
    )(combined, b0.reshape(1, H_DIM)[:, :RTC], W0)

    newpfc, alpha11, onehot, nm = pl.pallas_call(
        _tc2_kernel,
        in_specs=[pl.BlockSpec(memory_space=pltpu.VMEM)] * 14
        + [vm] * 6,
        out_specs=[pl.BlockSpec(memory_space=pltpu.VMEM)] * 4,
        out_shape=[
            jax.ShapeDtypeStruct((PFC_DIM, 1), f32),
            jax.ShapeDtypeStruct((1, 1), f32),
            jax.ShapeDtypeStruct((N_SLOTS, 1), f32),
            jax.ShapeDtypeStruct((3 * N_PATCHES, 1), f32),
        ],
        scratch_shapes=[
            pltpu.VMEM((NRING, BM, H_DIM), f32),
            pltpu.VMEM((PFC_DIM, D_MEM), f32),
            pltpu.VMEM((1, EPS, D_MEM), f32),
            pltpu.VMEM((1, EPS, 1), f32),
            pltpu.VMEM((1, 1, 1), jnp.int32),
            pltpu.VMEM((1, RSC), f32),
            pltpu.VMEM((1, KEY_DIM), f32),
            pltpu.SMEM((1,), f32),
            pltpu.SMEM((1,), jnp.int32),
            pltpu.SemaphoreType.DMA((NRING,)),
            pltpu.SemaphoreType.DMA,
        ],
    )(h_tc, h_sc.reshape(RSC, 16), b0.reshape(1, H_DIM)[:, RTC:],
      b2.reshape(1, KEY_DIM),
      pfc_state, pfc_state.reshape(PFC_DIM, 1),
      current_td_error.reshape(1, 1), Wg1, bg1.reshape(16, 1),
      Wg2.reshape(16, 1), bg2.reshape(1, 1), br.reshape(PFC_DIM, 1),
      Wn, bn.reshape(3 * N_PATCHES, 1),
      W2, prototypes, episodes, ep_td_errors.reshape(N_SLOTS, EPS, 1),
      ep_count.reshape(N_SLOTS, 1, 1), Wr)

    new_pfc = newpfc.reshape(1, PFC_DIM)
    alpha = alpha11.reshape(())
    one_hot_st = onehot.reshape(N_SLOTS)
    nmflat = nm.reshape(3 * N_PATCHES)
    eta = nmflat[0:N_PATCHES]
    decay = nmflat[N_PATCHES:2 * N_PATCHES]
    expl = nmflat[2 * N_PATCHES:]
    return (new_pfc, alpha, one_hot_st, eta, decay, expl)


# speedup vs baseline: 1.0133x; 1.0133x over previous
import functools

import jax
import jax.numpy as jnp
from jax import lax
from jax.experimental import pallas as pl
from jax.experimental.pallas import tpu as pltpu
from jax.experimental.pallas import tpu_sc as plsc

KEY_DIM = 4096
PFC_DIM = 1024
N_PATCHES = 4
N_SLOTS = 1024
EPS = 8
D_MEM = PFC_DIM + N_PATCHES * 3
IN_DIM = KEY_DIM + PFC_DIM
H_DIM = KEY_DIM * 2

RSC = 1536
RTC = H_DIM - RSC

BM = 256
NB_A = RTC // BM
NB_B = KEY_DIM // BM
NB_C = N_SLOTS // BM
NRING = 5

NW = 32
ROWS_W = RSC // NW
CH = 8
NCH = ROWS_W // CH

_NEG = float('-inf')


def _sc_h_body(w0_hbm, c_hbm, hp_hbm, cbuf, wchunk, obuf, sems):
    wid = lax.axis_index("s") * 2 + lax.axis_index("c")
    base = wid * ROWS_W

    def chunk_copy(ch, par):
        return pltpu.make_async_copy(
            w0_hbm.at[pl.ds(RTC + base + ch * CH, CH), :],
            wchunk.at[par], sems.at[par])

    chunk_copy(0, 0).start()
    pltpu.sync_copy(c_hbm, cbuf)

    for ch in range(NCH):
        par = ch % 2
        if ch + 1 < NCH:
            chunk_copy(ch + 1, (ch + 1) % 2).start()
        chunk_copy(ch, par).wait()
        for r in range(CH):
            def jbody(j, acc):
                sl = pl.ds(j * 16, 16)
                return acc + wchunk[par, r, sl] * cbuf[sl]

            acc = lax.fori_loop(0, IN_DIM // 16, jbody,
                                jnp.zeros((16,), jnp.float32))
            obuf[pl.ds((ch * CH + r) * 16, 16)] = acc

    pltpu.sync_copy(obuf, hp_hbm.at[pl.ds(base * 16, ROWS_W * 16)])


def _make_sc_h_kernel():
    return pl.kernel(
        _sc_h_body,
        mesh=plsc.VectorSubcoreMesh(core_axis_name="c", subcore_axis_name="s"),
        out_type=jax.ShapeDtypeStruct((RSC * 16,), jnp.float32),
        scratch_types=[
            pltpu.VMEM((IN_DIM,), jnp.float32),
            pltpu.VMEM((2, CH, IN_DIM), jnp.float32),
            pltpu.VMEM((ROWS_W * 16,), jnp.float32),
            pltpu.SemaphoreType.DMA((2,)),
        ])


def _tc1_kernel(comb_ref, b0_ref, w0_hbm, h_ref, wbuf, sems):

    def w0_copy(i, p):
        return pltpu.make_async_copy(
            w0_hbm.at[pl.ds(i * BM, BM), :], wbuf.at[p], sems.at[p])

    for g in range(NRING - 1):
        w0_copy(g, g % NRING).start()

    comb = comb_ref[...]

    def body_a(i, _):
        p = lax.rem(i, NRING)

        @pl.when(i + NRING - 1 < NB_A)
        def _():
            w0_copy(i + NRING - 1, lax.rem(i + NRING - 1, NRING)).start()

        w0_copy(i, p).wait()
        w = wbuf[p]
        col = jnp.sum(w * comb, axis=1, keepdims=True)
        h_ref[:, pl.ds(i * BM, BM)] = jnp.maximum(
            col.T + b0_ref[:, pl.ds(i * BM, BM)], 0.0)
        return 0

    lax.fori_loop(0, NB_A, body_a, 0)


def _tc2_kernel(htc_ref, hp_ref, b0sc_ref, b2_ref, pfc_row_ref, pfc_col_ref,
                tde_ref, wg1_ref, bg1_ref, wg2_ref, bg2_ref, br_ref,
                wn_ref, bn_ref,
                w2_hbm, proto_hbm, ep_hbm, td_hbm, cnt_hbm, wr_hbm,
                newpfc_ref, alpha_ref, onehot_ref, nm_ref,
                wbuf, wrbuf, epbuf, tdbuf, cntbuf, hsc_row, key_row,
                best_val, best_idx, sems, wrsem):

    NB2 = NB_B + NB_C

    def w2_copy(i, p):
        return pltpu.make_async_copy(
            w2_hbm.at[pl.ds(i * BM, BM), :], wbuf.at[p], sems.at[p])

    def proto_copy(i, p):
        return pltpu.make_async_copy(
            proto_hbm.at[pl.ds(i * BM, BM), :],
            wbuf.at[p, :, pl.ds(0, KEY_DIM)], sems.at[p])

    def start_block(g):
        p = lax.rem(g, NRING)

        @pl.when(g < NB_B)
        def _():
            w2_copy(g, p).start()

        @pl.when(jnp.logical_and(g >= NB_B, g < NB2))
        def _():
            proto_copy(g - NB_B, p).start()

    pltpu.make_async_copy(wr_hbm, wrbuf, wrsem).start()
    for g in range(NRING - 1):
        w2_copy(g, g % NRING).start()

    htc = htc_ref[...]

    for j in range(RSC // BM):
        colp = jnp.sum(hp_ref[pl.ds(j * BM, BM), :], axis=1, keepdims=True)
        hsc_row[:, pl.ds(j * BM, BM)] = jnp.maximum(
            colp.T + b0sc_ref[:, pl.ds(j * BM, BM)], 0.0)
    hsc = hsc_row[...]

    def body_b(i, _):
        p = lax.rem(i, NRING)
        start_block(i + NRING - 1)
        w2_copy(i, p).wait()
        col = jnp.sum(wbuf[p, :, pl.ds(0, RTC)] * htc, axis=1, keepdims=True) \
            + jnp.sum(wbuf[p, :, pl.ds(RTC, RSC)] * hsc, axis=1, keepdims=True)
        key_row[:, pl.ds(i * BM, BM)] = col.T + b2_ref[:, pl.ds(i * BM, BM)]
        return 0

    lax.fori_loop(0, NB_B, body_b, 0)

    best_val[0] = _NEG
    best_idx[0] = 0
    key = key_row[...]
    knorm = jnp.clip(jnp.sqrt(jnp.sum(key * key)), 1e-12, None)

    def body_c(i, _):
        g = NB_B + i
        p = lax.rem(g, NRING)
        proto_copy(i, p).wait()
        w = wbuf[p, :, pl.ds(0, KEY_DIM)]
        raw = jnp.sum(w * key, axis=1, keepdims=True)
        pn = jnp.clip(jnp.sqrt(jnp.sum(w * w, axis=1, keepdims=True)),
                      1e-12, None)
        sims = raw / (pn * knorm)
        mx = jnp.max(sims)
        idx = lax.broadcasted_iota(jnp.int32, (BM, 1), 0) + i * BM
        bidx = jnp.min(jnp.where(sims == mx, idx, jnp.int32(2**30)))

        @pl.when(mx > best_val[0])
        def _():
            best_val[0] = mx
            best_idx[0] = bidx

        return 0

    lax.fori_loop(0, NB_C, body_c, 0)

    slot = best_idx[0]
    ep_cp = pltpu.make_async_copy(ep_hbm.at[pl.ds(slot, 1)], epbuf, sems.at[0])
    td_cp = pltpu.make_async_copy(td_hbm.at[pl.ds(slot, 1)], tdbuf, sems.at[1])
    cnt_cp = pltpu.make_async_copy(cnt_hbm.at[pl.ds(slot, 1)], cntbuf,
                                   sems.at[2])
    ep_cp.start()
    td_cp.start()
    cnt_cp.start()
    ep_cp.wait()
    td_cp.wait()
    cnt_cp.wait()
    pltpu.make_async_copy(wr_hbm, wrbuf, wrsem).wait()

    eps = epbuf[0]
    stored = eps[:, :PFC_DIM]
    pfc_row = pfc_row_ref[...]
    pnorm = jnp.clip(jnp.sqrt(jnp.sum(pfc_row * pfc_row)), 1e-12, None)
    pn = pfc_row / pnorm
    snorm = jnp.clip(jnp.sqrt(jnp.sum(stored * stored, axis=1, keepdims=True)),
                     1e-12, None)
    sims_e = jnp.sum(stored * pn, axis=1, keepdims=True) / snorm

    td = tdbuf[0]
    rel = sims_e * jnp.clip(jnp.abs(td), 1e-6, None)
    n_eps = jnp.minimum(cntbuf[0], EPS)
    idx8 = lax.broadcasted_iota(jnp.int32, (EPS, 1), 0)
    rel = jnp.where(idx8 < n_eps, rel, _NEG)
    mx = jnp.max(rel, axis=0, keepdims=True)
    bidx = jnp.min(jnp.where(rel == mx, idx8, jnp.int32(2**30)),
                   axis=0, keepdims=True)
    oh8 = (idx8 == bidx).astype(jnp.float32)
    ep_content = jnp.sum(eps * oh8, axis=0, keepdims=True)
    ep_td = jnp.sum(td * oh8, axis=0, keepdims=True)

    wg1 = wg1_ref[...]
    x1 = jnp.abs(tde_ref[...])
    g = jnp.tanh(wg1[:, 0:1] * best_val[0] + wg1[:, 1:2] * x1
                 + wg1[:, 2:3] * ep_td + bg1_ref[...])
    alpha = jnp.tanh(jnp.sum(wg2_ref[...] * g, axis=0, keepdims=True)
                     + bg2_ref[...])

    delta = jnp.sum(wrbuf[...] * ep_content, axis=1, keepdims=True) \
        + br_ref[...]
    newpfc_ref[...] = pfc_col_ref[...] + alpha * delta
    alpha_ref[...] = alpha

    ii = lax.broadcasted_iota(jnp.int32, (N_SLOTS, 1), 0)
    onehot_ref[...] = (ii == slot).astype(jnp.float32)

    nm = jnp.sum(wn_ref[...] * ep_content, axis=1, keepdims=True) + bn_ref[...]
    rows = lax.broadcasted_iota(jnp.int32, (3 * N_PATCHES, 1), 0)
    hi = jnp.where(rows < 2 * N_PATCHES, 1.0, 0.5)
    nm_ref[...] = jnp.clip(nm, 0.1, hi)


def kernel(activation_summary, pfc_state, current_td_error, prototypes,
           log_temperature, W0, b0, W2, b2, episodes, ep_td_errors, ep_count,
           Wg1, bg1, Wg2, bg2, Wr, br, Wn, bn):
    f32 = jnp.float32
    combined = jnp.concatenate(
        [activation_summary.reshape(1, KEY_DIM), pfc_state], axis=1)

    h_sc = _make_sc_h_kernel()(W0, combined.reshape(IN_DIM))

    vm = pl.BlockSpec(memory_space=pl.ANY)
    h_tc = pl.pallas_call(
        _tc1_kernel,
        in_specs=[pl.BlockSpec(memory_space=pltpu.VMEM),
                  pl.BlockSpec(memory_space=pltpu.VMEM), vm],
        out_specs=pl.BlockSpec(memory_space=pltpu.VMEM),
        out_shape=jax.ShapeDtypeStruct((1, RTC), f32),
        scratch_shapes=[
            pltpu.VMEM((NRING, BM, IN_DIM), f32),
            pltpu.SemaphoreType.DMA((NRING,)),
        ],
    )(combined, b0.reshape(1, H_DIM)[:, :RTC], W0)

    newpfc, alpha11, onehot, nm = pl.pallas_call(
        _tc2_kernel,
        in_specs=[pl.BlockSpec(memory_space=pltpu.VMEM)] * 14
        + [vm] * 6,
        out_specs=[pl.BlockSpec(memory_space=pltpu.VMEM)] * 4,
        out_shape=[
            jax.ShapeDtypeStruct((PFC_DIM, 1), f32),
            jax.ShapeDtypeStruct((1, 1), f32),
            jax.ShapeDtypeStruct((N_SLOTS, 1), f32),
            jax.ShapeDtypeStruct((3 * N_PATCHES, 1), f32),
        ],
        scratch_shapes=[
            pltpu.VMEM((NRING, BM, H_DIM), f32),
            pltpu.VMEM((PFC_DIM, D_MEM), f32),
            pltpu.VMEM((1, EPS, D_MEM), f32),
            pltpu.VMEM((1, EPS, 1), f32),
            pltpu.VMEM((1, 1, 1), jnp.int32),
            pltpu.VMEM((1, RSC), f32),
            pltpu.VMEM((1, KEY_DIM), f32),
            pltpu.SMEM((1,), f32),
            pltpu.SMEM((1,), jnp.int32),
            pltpu.SemaphoreType.DMA((NRING,)),
            pltpu.SemaphoreType.DMA,
        ],
    )(h_tc, h_sc.reshape(RSC, 16), b0.reshape(1, H_DIM)[:, RTC:],
      b2.reshape(1, KEY_DIM),
      pfc_state, pfc_state.reshape(PFC_DIM, 1),
      current_td_error.reshape(1, 1), Wg1, bg1.reshape(16, 1),
      Wg2.reshape(16, 1), bg2.reshape(1, 1), br.reshape(PFC_DIM, 1),
      Wn, bn.reshape(3 * N_PATCHES, 1),
      W2, prototypes, episodes, ep_td_errors.reshape(N_SLOTS, EPS, 1),
      ep_count.reshape(N_SLOTS, 1, 1), Wr)

    new_pfc = newpfc.reshape(1, PFC_DIM)
    alpha = alpha11.reshape(())
    one_hot_st = onehot.reshape(N_SLOTS)
    nmflat = nm.reshape(3 * N_PATCHES)
    eta = nmflat[0:N_PATCHES]
    decay = nmflat[N_PATCHES:2 * N_PATCHES]
    expl = nmflat[2 * N_PATCHES:]
    return (new_pfc, alpha, one_hot_st, eta, decay, expl)


# speedup vs baseline: 1.0347x; 1.0211x over previous
import functools

import jax
import jax.numpy as jnp
from jax import lax
from jax.experimental import pallas as pl
from jax.experimental.pallas import tpu as pltpu
from jax.experimental.pallas import tpu_sc as plsc

KEY_DIM = 4096
PFC_DIM = 1024
N_PATCHES = 4
N_SLOTS = 1024
EPS = 8
D_MEM = PFC_DIM + N_PATCHES * 3
IN_DIM = KEY_DIM + PFC_DIM
H_DIM = KEY_DIM * 2

RSC = 2048
RTC = H_DIM - RSC

BM = 256
NB_A = RTC // BM
NB_B = KEY_DIM // BM
NB_C = N_SLOTS // BM
NRING = 5

NW = 32
ROWS_W = RSC // NW
CH = 8
NCH = ROWS_W // CH

_NEG = float('-inf')


def _sc_h_body(w0_hbm, c_hbm, hp_hbm, cbuf, wchunk, obuf, sems):
    wid = lax.axis_index("s") * 2 + lax.axis_index("c")
    base = wid * ROWS_W

    def chunk_copy(ch, par):
        return pltpu.make_async_copy(
            w0_hbm.at[pl.ds(RTC + base + ch * CH, CH), :],
            wchunk.at[par], sems.at[par])

    chunk_copy(0, 0).start()
    pltpu.sync_copy(c_hbm, cbuf)

    for ch in range(NCH):
        par = ch % 2
        if ch + 1 < NCH:
            chunk_copy(ch + 1, (ch + 1) % 2).start()
        chunk_copy(ch, par).wait()
        for r in range(CH):
            def jbody(j, acc):
                sl = pl.ds(j * 16, 16)
                return acc + wchunk[par, r, sl] * cbuf[sl]

            acc = lax.fori_loop(0, IN_DIM // 16, jbody,
                                jnp.zeros((16,), jnp.float32))
            obuf[pl.ds((ch * CH + r) * 16, 16)] = acc

    pltpu.sync_copy(obuf, hp_hbm.at[pl.ds(base * 16, ROWS_W * 16)])


def _make_sc_h_kernel():
    return pl.kernel(
        _sc_h_body,
        mesh=plsc.VectorSubcoreMesh(core_axis_name="c", subcore_axis_name="s"),
        out_type=jax.ShapeDtypeStruct((RSC * 16,), jnp.float32),
        scratch_types=[
            pltpu.VMEM((IN_DIM,), jnp.float32),
            pltpu.VMEM((2, CH, IN_DIM), jnp.float32),
            pltpu.VMEM((ROWS_W * 16,), jnp.float32),
            pltpu.SemaphoreType.DMA((2,)),
        ])


def _tc1_kernel(comb_ref, b0_ref, w0_hbm, h_ref, wbuf, sems):

    def w0_copy(i, p):
        return pltpu.make_async_copy(
            w0_hbm.at[pl.ds(i * BM, BM), :], wbuf.at[p], sems.at[p])

    for g in range(NRING - 1):
        w0_copy(g, g % NRING).start()

    comb = comb_ref[...]

    def body_a(i, _):
        p = lax.rem(i, NRING)

        @pl.when(i + NRING - 1 < NB_A)
        def _():
            w0_copy(i + NRING - 1, lax.rem(i + NRING - 1, NRING)).start()

        w0_copy(i, p).wait()
        w = wbuf[p]
        col = jnp.sum(w * comb, axis=1, keepdims=True)
        h_ref[:, pl.ds(i * BM, BM)] = jnp.maximum(
            col.T + b0_ref[:, pl.ds(i * BM, BM)], 0.0)
        return 0

    lax.fori_loop(0, NB_A, body_a, 0)


def _tc2_kernel(htc_ref, hp_ref, b0sc_ref, b2_ref, pfc_row_ref, pfc_col_ref,
                tde_ref, wg1_ref, bg1_ref, wg2_ref, bg2_ref, br_ref,
                wn_ref, bn_ref,
                w2_hbm, proto_hbm, ep_hbm, td_hbm, cnt_hbm, wr_hbm,
                newpfc_ref, alpha_ref, onehot_ref, nm_ref,
                wbuf, wrbuf, epbuf, tdbuf, cntbuf, hsc_row, key_row,
                best_val, best_idx, sems, wrsem):

    NB2 = NB_B + NB_C

    def w2_copy(i, p):
        return pltpu.make_async_copy(
            w2_hbm.at[pl.ds(i * BM, BM), :], wbuf.at[p], sems.at[p])

    def proto_copy(i, p):
        return pltpu.make_async_copy(
            proto_hbm.at[pl.ds(i * BM, BM), :],
            wbuf.at[p, :, pl.ds(0, KEY_DIM)], sems.at[p])

    def start_block(g):
        p = lax.rem(g, NRING)

        @pl.when(g < NB_B)
        def _():
            w2_copy(g, p).start()

        @pl.when(jnp.logical_and(g >= NB_B, g < NB2))
        def _():
            proto_copy(g - NB_B, p).start()

    pltpu.make_async_copy(wr_hbm, wrbuf, wrsem).start()
    for g in range(NRING - 1):
        w2_copy(g, g % NRING).start()

    htc = htc_ref[...]

    for j in range(RSC // BM):
        colp = jnp.sum(hp_ref[pl.ds(j * BM, BM), :], axis=1, keepdims=True)
        hsc_row[:, pl.ds(j * BM, BM)] = jnp.maximum(
            colp.T + b0sc_ref[:, pl.ds(j * BM, BM)], 0.0)
    hsc = hsc_row[...]

    def body_b(i, _):
        p = lax.rem(i, NRING)
        start_block(i + NRING - 1)
        w2_copy(i, p).wait()
        col = jnp.sum(wbuf[p, :, pl.ds(0, RTC)] * htc, axis=1, keepdims=True) \
            + jnp.sum(wbuf[p, :, pl.ds(RTC, RSC)] * hsc, axis=1, keepdims=True)
        key_row[:, pl.ds(i * BM, BM)] = col.T + b2_ref[:, pl.ds(i * BM, BM)]
        return 0

    lax.fori_loop(0, NB_B, body_b, 0)

    best_val[0] = _NEG
    best_idx[0] = 0
    key = key_row[...]
    knorm = jnp.clip(jnp.sqrt(jnp.sum(key * key)), 1e-12, None)

    def body_c(i, _):
        g = NB_B + i
        p = lax.rem(g, NRING)
        proto_copy(i, p).wait()
        w = wbuf[p, :, pl.ds(0, KEY_DIM)]
        raw = jnp.sum(w * key, axis=1, keepdims=True)
        pn = jnp.clip(jnp.sqrt(jnp.sum(w * w, axis=1, keepdims=True)),
                      1e-12, None)
        sims = raw / (pn * knorm)
        mx = jnp.max(sims)
        idx = lax.broadcasted_iota(jnp.int32, (BM, 1), 0) + i * BM
        bidx = jnp.min(jnp.where(sims == mx, idx, jnp.int32(2**30)))

        @pl.when(mx > best_val[0])
        def _():
            best_val[0] = mx
            best_idx[0] = bidx

        return 0

    lax.fori_loop(0, NB_C, body_c, 0)

    slot = best_idx[0]
    ep_cp = pltpu.make_async_copy(ep_hbm.at[pl.ds(slot, 1)], epbuf, sems.at[0])
    td_cp = pltpu.make_async_copy(td_hbm.at[pl.ds(slot, 1)], tdbuf, sems.at[1])
    cnt_cp = pltpu.make_async_copy(cnt_hbm.at[pl.ds(slot, 1)], cntbuf,
                                   sems.at[2])
    ep_cp.start()
    td_cp.start()
    cnt_cp.start()
    ep_cp.wait()
    td_cp.wait()
    cnt_cp.wait()
    pltpu.make_async_copy(wr_hbm, wrbuf, wrsem).wait()

    eps = epbuf[0]
    stored = eps[:, :PFC_DIM]
    pfc_row = pfc_row_ref[...]
    pnorm = jnp.clip(jnp.sqrt(jnp.sum(pfc_row * pfc_row)), 1e-12, None)
    pn = pfc_row / pnorm
    snorm = jnp.clip(jnp.sqrt(jnp.sum(stored * stored, axis=1, keepdims=True)),
                     1e-12, None)
    sims_e = jnp.sum(stored * pn, axis=1, keepdims=True) / snorm

    td = tdbuf[0]
    rel = sims_e * jnp.clip(jnp.abs(td), 1e-6, None)
    n_eps = jnp.minimum(cntbuf[0], EPS)
    idx8 = lax.broadcasted_iota(jnp.int32, (EPS, 1), 0)
    rel = jnp.where(idx8 < n_eps, rel, _NEG)
    mx = jnp.max(rel, axis=0, keepdims=True)
    bidx = jnp.min(jnp.where(rel == mx, idx8, jnp.int32(2**30)),
                   axis=0, keepdims=True)
    oh8 = (idx8 == bidx).astype(jnp.float32)
    ep_content = jnp.sum(eps * oh8, axis=0, keepdims=True)
    ep_td = jnp.sum(td * oh8, axis=0, keepdims=True)

    wg1 = wg1_ref[...]
    x1 = jnp.abs(tde_ref[...])
    g = jnp.tanh(wg1[:, 0:1] * best_val[0] + wg1[:, 1:2] * x1
                 + wg1[:, 2:3] * ep_td + bg1_ref[...])
    alpha = jnp.tanh(jnp.sum(wg2_ref[...] * g, axis=0, keepdims=True)
                     + bg2_ref[...])

    delta = jnp.sum(wrbuf[...] * ep_content, axis=1, keepdims=True) \
        + br_ref[...]
    newpfc_ref[...] = pfc_col_ref[...] + alpha * delta
    alpha_ref[...] = alpha

    ii = lax.broadcasted_iota(jnp.int32, (N_SLOTS, 1), 0)
    onehot_ref[...] = (ii == slot).astype(jnp.float32)

    nm = jnp.sum(wn_ref[...] * ep_content, axis=1, keepdims=True) + bn_ref[...]
    rows = lax.broadcasted_iota(jnp.int32, (3 * N_PATCHES, 1), 0)
    hi = jnp.where(rows < 2 * N_PATCHES, 1.0, 0.5)
    nm_ref[...] = jnp.clip(nm, 0.1, hi)


def kernel(activation_summary, pfc_state, current_td_error, prototypes,
           log_temperature, W0, b0, W2, b2, episodes, ep_td_errors, ep_count,
           Wg1, bg1, Wg2, bg2, Wr, br, Wn, bn):
    f32 = jnp.float32
    combined = jnp.concatenate(
        [activation_summary.reshape(1, KEY_DIM), pfc_state], axis=1)

    h_sc = _make_sc_h_kernel()(W0, combined.reshape(IN_DIM))

    vm = pl.BlockSpec(memory_space=pl.ANY)
    h_tc = pl.pallas_call(
        _tc1_kernel,
        in_specs=[pl.BlockSpec(memory_space=pltpu.VMEM),
                  pl.BlockSpec(memory_space=pltpu.VMEM), vm],
        out_specs=pl.BlockSpec(memory_space=pltpu.VMEM),
        out_shape=jax.ShapeDtypeStruct((1, RTC), f32),
        scratch_shapes=[
            pltpu.VMEM((NRING, BM, IN_DIM), f32),
            pltpu.SemaphoreType.DMA((NRING,)),
        ],
    )(combined, b0.reshape(1, H_DIM)[:, :RTC], W0)

    newpfc, alpha11, onehot, nm = pl.pallas_call(
        _tc2_kernel,
        in_specs=[pl.BlockSpec(memory_space=pltpu.VMEM)] * 14
        + [vm] * 6,
        out_specs=[pl.BlockSpec(memory_space=pltpu.VMEM)] * 4,
        out_shape=[
            jax.ShapeDtypeStruct((PFC_DIM, 1), f32),
            jax.ShapeDtypeStruct((1, 1), f32),
            jax.ShapeDtypeStruct((N_SLOTS, 1), f32),
            jax.ShapeDtypeStruct((3 * N_PATCHES, 1), f32),
        ],
        scratch_shapes=[
            pltpu.VMEM((NRING, BM, H_DIM), f32),
            pltpu.VMEM((PFC_DIM, D_MEM), f32),
            pltpu.VMEM((1, EPS, D_MEM), f32),
            pltpu.VMEM((1, EPS, 1), f32),
            pltpu.VMEM((1, 1, 1), jnp.int32),
            pltpu.VMEM((1, RSC), f32),
            pltpu.VMEM((1, KEY_DIM), f32),
            pltpu.SMEM((1,), f32),
            pltpu.SMEM((1,), jnp.int32),
            pltpu.SemaphoreType.DMA((NRING,)),
            pltpu.SemaphoreType.DMA,
        ],
    )(h_tc, h_sc.reshape(RSC, 16), b0.reshape(1, H_DIM)[:, RTC:],
      b2.reshape(1, KEY_DIM),
      pfc_state, pfc_state.reshape(PFC_DIM, 1),
      current_td_error.reshape(1, 1), Wg1, bg1.reshape(16, 1),
      Wg2.reshape(16, 1), bg2.reshape(1, 1), br.reshape(PFC_DIM, 1),
      Wn, bn.reshape(3 * N_PATCHES, 1),
      W2, prototypes, episodes, ep_td_errors.reshape(N_SLOTS, EPS, 1),
      ep_count.reshape(N_SLOTS, 1, 1), Wr)

    new_pfc = newpfc.reshape(1, PFC_DIM)
    alpha = alpha11.reshape(())
    one_hot_st = onehot.reshape(N_SLOTS)
    nmflat = nm.reshape(3 * N_PATCHES)
    eta = nmflat[0:N_PATCHES]
    decay = nmflat[N_PATCHES:2 * N_PATCHES]
    expl = nmflat[2 * N_PATCHES:]
    return (new_pfc, alpha, one_hot_st, eta, decay, expl)
